# Initial kernel scaffold; baseline (speedup 1.0000x reference)
import jax
import jax.numpy as jnp
from jax.experimental import pallas as pl


def kernel(x, edge_index, edge_attr, batch, select_bond_start_atom_index, select_bond_end_atom_index, atom_W, atom_b, edge_W, edge_b, mlp_W1, mlp_b1, mlp_W2, mlp_b2, eps, out_W, out_b, pp_W1, pp_b1, pp_W2, pp_b2, pr_W1, pr_b1, pr_W2, pr_b2, pr_W3, pr_b3):
    raise NotImplementedError("write your pallas kernel here")


# speedup vs baseline: 2.3745x; 2.3745x over previous
import functools

import jax
import jax.numpy as jnp
from jax import lax
from jax.experimental import pallas as pl
from jax.experimental.pallas import tpu as pltpu
from jax.experimental.pallas import tpu_sc as plsc

F32 = jnp.float32


def _silu(v):
    return v * jax.nn.sigmoid(v)


def _matmul_bias(x, w, b, block_rows):
    m, k = x.shape
    h = w.shape[1]
    grid = m // block_rows

    def body(x_ref, w_ref, b_ref, o_ref):
        o_ref[...] = (
            jnp.dot(x_ref[...], w_ref[...], preferred_element_type=F32)
            + b_ref[...]
        )

    return pl.pallas_call(
        body,
        grid=(grid,),
        in_specs=[
            pl.BlockSpec((block_rows, k), lambda i: (i, 0)),
            pl.BlockSpec((k, h), lambda i: (0, 0)),
            pl.BlockSpec((1, h), lambda i: (0, 0)),
        ],
        out_specs=pl.BlockSpec((block_rows, h), lambda i: (i, 0)),
        out_shape=jax.ShapeDtypeStruct((m, h), F32),
    )(x, w, b.reshape(1, -1))


def _node_mlp(h, agg2, scale_row, w1, b1, w2, b2, w3, b3, block_rows):
    m, hh = h.shape
    grid = m // block_rows
    apply_out = w3 is not None

    def body(h_ref, a0_ref, a1_ref, s_ref, w1_ref, b1_ref, w2_ref, b2_ref,
             *rest):
        o_ref = rest[-1]
        z = h_ref[...] * s_ref[...] + a0_ref[...] + a1_ref[...]
        t = jnp.dot(z, w1_ref[...], preferred_element_type=F32) + b1_ref[...]
        t = _silu(t)
        u = jnp.dot(t, w2_ref[...], preferred_element_type=F32) + b2_ref[...]
        u = _silu(u)
        if apply_out:
            u = (jnp.dot(u, rest[0][...], preferred_element_type=F32)
                 + rest[1][...])
        o_ref[...] = u

    in_specs = [
        pl.BlockSpec((block_rows, hh), lambda i: (i, 0)),
        pl.BlockSpec((block_rows, hh), lambda i: (i, 0)),
        pl.BlockSpec((block_rows, hh), lambda i, g=grid: (i + g, 0)),
        pl.BlockSpec((1, hh), lambda i: (0, 0)),
        pl.BlockSpec((hh, hh), lambda i: (0, 0)),
        pl.BlockSpec((1, hh), lambda i: (0, 0)),
        pl.BlockSpec((hh, hh), lambda i: (0, 0)),
        pl.BlockSpec((1, hh), lambda i: (0, 0)),
    ]
    args = [h, agg2, agg2, scale_row, w1, b1.reshape(1, -1), w2,
            b2.reshape(1, -1)]
    if apply_out:
        in_specs += [
            pl.BlockSpec((hh, hh), lambda i: (0, 0)),
            pl.BlockSpec((1, hh), lambda i: (0, 0)),
        ]
        args += [w3, b3.reshape(1, -1)]

    return pl.pallas_call(
        body,
        grid=(grid,),
        in_specs=in_specs,
        out_specs=pl.BlockSpec((block_rows, hh), lambda i: (i, 0)),
        out_shape=jax.ShapeDtypeStruct((m, hh), F32),
    )(*args)


def _predictor(hs, he, hagg2, pp_w1, pp_b1, pp_w2, pp_b2,
               pr_w1, pr_b1, pr_w2, pr_b2, pr_w3p, pr_b3p):
    g, hh = hs.shape

    def body(hs_ref, he_ref, hagg_ref, pw1, pb1, pw2, pb2,
             rw1, rb1, rw2, rb2, rw3, rb3, o_ref):
        hagg = hagg_ref[:g, :] + hagg_ref[g:, :]
        a = jnp.concatenate([hs_ref[...], he_ref[...]], axis=1)
        b = jnp.concatenate([he_ref[...], hs_ref[...]], axis=1)

        def pp(v):
            t = _silu(jnp.dot(v, pw1[...], preferred_element_type=F32)
                      + pb1[...])
            return jnp.dot(t, pw2[...], preferred_element_type=F32) + pb2[...]

        h_bond = pp(a) + pp(b)
        hout = jnp.concatenate([h_bond, hagg], axis=1)
        t = _silu(jnp.dot(hout, rw1[...], preferred_element_type=F32)
                  + rb1[...])
        t = _silu(jnp.dot(t, rw2[...], preferred_element_type=F32)
                  + rb2[...])
        o_ref[...] = (jnp.dot(t, rw3[...], preferred_element_type=F32)
                      + rb3[...])

    return pl.pallas_call(
        body,
        out_shape=jax.ShapeDtypeStruct((g, hh), F32),
    )(hs, he, hagg2, pp_w1, pp_b1.reshape(1, -1), pp_w2, pp_b2.reshape(1, -1),
      pr_w1, pr_b1.reshape(1, -1), pr_w2, pr_b2.reshape(1, -1),
      pr_w3p, pr_b3p.reshape(1, -1))


_NC = 2
_NS = 16
_NW = _NC * _NS


def _make_sc_message(n_nodes, n_edges, hh):
    epw = n_edges // _NW
    ch = 80
    steps = epw // ch
    rstride = ((n_nodes // _NS) // 8) * 8
    rlast = n_nodes - (_NS - 1) * rstride
    nj = hh // 16

    mesh = plsc.VectorSubcoreMesh(core_axis_name="c", subcore_axis_name="s",
                                  num_cores=_NC, num_subcores=_NS)

    @functools.partial(
        pl.kernel,
        mesh=mesh,
        out_type=jax.ShapeDtypeStruct((2 * n_nodes, hh), F32),
        scratch_types=[
            pltpu.VMEM((ch,), jnp.int32),
            pltpu.VMEM((ch,), jnp.int32),
            pltpu.VMEM((ch, hh), F32),
            pltpu.VMEM((ch, hh), F32),
            pltpu.VMEM_SHARED((n_nodes, hh), F32),
            pltpu.SemaphoreType.DMA,
        ],
    )
    def k(h_hbm, src_hbm, dst_hbm, e_hbm, out_hbm,
          sidx, didx, rows, ebuf, acc, sem):
        c = lax.axis_index("c")
        s = lax.axis_index("s")
        wid = c * _NS + s

        def zbody(i, _):
            for j in range(nj):
                rows[i, pl.ds(j * 16, 16)] = jnp.zeros((16,), F32)
            return 0

        lax.fori_loop(0, ch, zbody, 0)
        for r in range(rlast // ch):
            pltpu.sync_copy(rows, acc.at[pl.ds(s * rstride + r * ch, ch)])
        plsc.subcore_barrier()

        base0 = wid * epw

        def step(t, _):
            base = base0 + t * ch
            pltpu.sync_copy(src_hbm.at[pl.ds(base, ch)], sidx)
            pltpu.sync_copy(dst_hbm.at[pl.ds(base, ch)], didx)
            pltpu.async_copy(h_hbm.at[sidx], rows, sem).wait()
            pltpu.sync_copy(e_hbm.at[pl.ds(base, ch)], ebuf)

            def comp(i, __):
                for j in range(nj):
                    sl = pl.ds(j * 16, 16)
                    rows[i, sl] = jnp.maximum(rows[i, sl] + ebuf[i, sl], 0.0)
                return 0

            lax.fori_loop(0, ch, comp, 0)
            pltpu.sync_copy(rows, acc.at[didx], add=True)
            return 0

        lax.fori_loop(0, steps, step, 0)
        plsc.subcore_barrier()

        @pl.when(s < _NS - 1)
        def _():
            pltpu.sync_copy(
                acc.at[pl.ds(s * rstride, rstride)],
                out_hbm.at[pl.ds(c * n_nodes + s * rstride, rstride)])

        @pl.when(s == _NS - 1)
        def _():
            off = (_NS - 1) * rstride
            pltpu.sync_copy(
                acc.at[pl.ds(off, rlast)],
                out_hbm.at[pl.ds(c * n_nodes + off, rlast)])

    return k


def _make_sc_readout(n_nodes, g, hh):
    n_pad = ((n_nodes + _NW * 64 - 1) // (_NW * 64)) * (_NW * 64)
    npw = n_pad // _NW
    ch = 64
    steps = npw // ch
    gpw = g // _NW
    zpt = ((g + 1 + 8 * _NS - 1) // (8 * _NS)) * 8
    nacc = zpt * _NS
    cpt = g // _NS
    nj = hh // 16

    mesh = plsc.VectorSubcoreMesh(core_axis_name="c", subcore_axis_name="s",
                                  num_cores=_NC, num_subcores=_NS)

    @functools.partial(
        pl.kernel,
        mesh=mesh,
        out_type=(
            jax.ShapeDtypeStruct((g, hh), F32),
            jax.ShapeDtypeStruct((g, hh), F32),
            jax.ShapeDtypeStruct((2 * g, hh), F32),
        ),
        scratch_types=[
            pltpu.VMEM((gpw,), jnp.int32),
            pltpu.VMEM((gpw, hh), F32),
            pltpu.VMEM((ch,), jnp.int32),
            pltpu.VMEM((ch,), jnp.int32),
            pltpu.VMEM((ch, hh), F32),
            pltpu.VMEM_SHARED((nacc, hh), F32),
            pltpu.SemaphoreType.DMA,
        ],
    )
    def k(h_hbm, sidx_hbm, eidx_hbm, nidx_hbm, batch_hbm,
          hs_hbm, he_hbm, hagg_hbm,
          bidx, brows, nidx, gids, hrows, acc, sem):
        c = lax.axis_index("c")
        s = lax.axis_index("s")
        wid = c * _NS + s

        def zbody(i, _):
            for j in range(nj):
                hrows[i, pl.ds(j * 16, 16)] = jnp.zeros((16,), F32)
            return 0

        lax.fori_loop(0, ch, zbody, 0)
        pltpu.sync_copy(hrows.at[pl.ds(0, zpt)], acc.at[pl.ds(s * zpt, zpt)])
        plsc.subcore_barrier()

        gb = wid * gpw
        pltpu.sync_copy(sidx_hbm.at[pl.ds(gb, gpw)], bidx)
        pltpu.async_copy(h_hbm.at[bidx], brows, sem).wait()
        pltpu.sync_copy(brows, hs_hbm.at[pl.ds(gb, gpw)])
        pltpu.sync_copy(eidx_hbm.at[pl.ds(gb, gpw)], bidx)
        pltpu.async_copy(h_hbm.at[bidx], brows, sem).wait()
        pltpu.sync_copy(brows, he_hbm.at[pl.ds(gb, gpw)])

        base0 = wid * npw

        def step(t, _):
            base = base0 + t * ch
            pltpu.sync_copy(nidx_hbm.at[pl.ds(base, ch)], nidx)
            pltpu.async_copy(h_hbm.at[nidx], hrows, sem).wait()
            pltpu.sync_copy(batch_hbm.at[pl.ds(base, ch)], gids)
            pltpu.sync_copy(hrows, acc.at[gids], add=True)
            return 0

        lax.fori_loop(0, steps, step, 0)
        plsc.subcore_barrier()
        pltpu.sync_copy(acc.at[pl.ds(s * cpt, cpt)],
                        hagg_hbm.at[pl.ds(c * g + s * cpt, cpt)])

    return k


def kernel(x, edge_index, edge_attr, batch, select_bond_start_atom_index,
           select_bond_end_atom_index, atom_W, atom_b, edge_W, edge_b,
           mlp_W1, mlp_b1, mlp_W2, mlp_b2, eps, out_W, out_b,
           pp_W1, pp_b1, pp_W2, pp_b2, pr_W1, pr_b1, pr_W2, pr_b2,
           pr_W3, pr_b3):
    n, _ = x.shape
    e = edge_index.shape[1]
    g = select_bond_start_atom_index.shape[0]
    hh = atom_W.shape[1]
    nl = edge_W.shape[0]

    src = edge_index[0]
    dst = edge_index[1]

    sc_message = _make_sc_message(n, e, hh)
    sc_readout = _make_sc_readout(n, g, hh)

    n_pad = ((n + _NW * 64 - 1) // (_NW * 64)) * (_NW * 64)
    ar = jnp.arange(n_pad, dtype=jnp.int32)
    nidx_pad = jnp.where(ar < n, ar, 0)
    batch_pad = jnp.concatenate(
        [batch, jnp.full((n_pad - n,), g, jnp.int32)])

    h = _matmul_bias(x, atom_W, atom_b, block_rows=n // 5)
    ones_row = jnp.ones((1, hh), F32)

    for l in range(nl):
        e_feat = _matmul_bias(edge_attr, edge_W[l], edge_b[l],
                              block_rows=e // 40)
        agg2 = sc_message(h, src, dst, e_feat)
        scale_row = (1.0 + eps[l]) * ones_row
        last = l == nl - 1
        h = _node_mlp(h, agg2, scale_row, mlp_W1[l], mlp_b1[l],
                      mlp_W2[l], mlp_b2[l],
                      out_W if last else None, out_b if last else None,
                      block_rows=n // 5)

    hs, he, hagg2 = sc_readout(h, select_bond_start_atom_index,
                               select_bond_end_atom_index, nidx_pad,
                               batch_pad)

    pr_w3p = jnp.pad(pr_W3, ((0, 0), (0, hh - pr_W3.shape[1])))
    pr_b3p = jnp.pad(pr_b3, (0, hh - pr_b3.shape[0]))
    outp = _predictor(hs, he, hagg2, pp_W1, pp_b1, pp_W2, pp_b2,
                      pr_W1, pr_b1, pr_W2, pr_b2, pr_w3p, pr_b3p)
    out = outp[:, :1]
    return out, h
